# baseline (speedup 1.0000x reference)
import functools
import jax
import jax.numpy as jnp
from jax import lax
from jax.experimental import pallas as pl
from jax.experimental.pallas import tpu as pltpu
from jax.experimental.pallas import tpu_sc as plsc

NT = 10000
NO = 2048
ET = 160000
EO = 65536
NB = 64
NA = 3129
D = 128
CT = ET // 128
CTP = 1256
CTG = 1264
GPC = 79
CO = EO // 128
TPT = 88
TQ = 1000
NSTEP = NT // TQ

F32 = jnp.float32


def _deg_body(tdst, odst, ones16, zz16, deg_t, deg_o, tidx, oidx, ones_v,
              acc_t, acc_o):
    c = lax.axis_index("c")
    s = lax.axis_index("s")
    pltpu.sync_copy(zz16, acc_t.at[pl.ds(s * 624, 624)])
    pltpu.sync_copy(zz16.at[pl.ds(0, 128)], acc_o.at[pl.ds(s * 128, 128)])

    @pl.when(s == 15)
    def _ztail():
        pltpu.sync_copy(zz16.at[pl.ds(0, 16)], acc_t.at[pl.ds(9984, 16)])

    pltpu.sync_copy(ones16, ones_v)
    plsc.subcore_barrier()
    lo = s * CT // 16
    hi = (s + 1) * CT // 16
    lo8 = (lo // 8) * 8
    pltpu.sync_copy(tdst.at[c, pl.ds(lo8, TPT)], tidx)
    pltpu.sync_copy(odst.at[pl.ds(c * (CO // 2) + s * 16, 16)], oidx)

    def tb(j, carry):
        pltpu.sync_copy(ones_v, acc_t.at[tidx.at[j]], add=True)
        return carry

    lax.fori_loop(lo - lo8, hi - lo8, tb, 0)

    def ob(j, carry):
        pltpu.sync_copy(ones_v, acc_o.at[oidx.at[j]], add=True)
        return carry

    lax.fori_loop(0, 16, ob, 0)
    plsc.subcore_barrier()
    pltpu.sync_copy(acc_t.at[pl.ds(s * 624, 624)],
                    deg_t.at[pl.ds(c * NT + s * 624, 624)])
    pltpu.sync_copy(acc_o.at[pl.ds(s * 128, 128)],
                    deg_o.at[pl.ds(c * NO + s * 128, 128)])

    @pl.when(s == 15)
    def _otail():
        pltpu.sync_copy(acc_t.at[pl.ds(9984, 16)],
                        deg_t.at[pl.ds(c * NT + 9984, 16)])


def _scatter_graph_body(y_txt, gsrc, gdst, zz, z2,
                        tsrc_v, tdst_v, rows_a, rows_b, acc_t,
                        sga, sgb, ssa, ssb):
    c = lax.axis_index("c")
    s = lax.axis_index("s")
    pltpu.sync_copy(zz, acc_t.at[pl.ds(s * 624, 624)])

    @pl.when(s == 15)
    def _ztail():
        pltpu.sync_copy(zz.at[pl.ds(0, 24)], acc_t.at[pl.ds(9984, 24)])

    plsc.subcore_barrier()
    g_lo = s * GPC // 16
    g_hi = (s + 1) * GPC // 16

    def fire(idxrow, buf, sm):
        pltpu.async_copy(y_txt.at[idxrow], buf, sm)

    def wait(buf, sm):
        pltpu.make_async_copy(y_txt.at[pl.ds(0, 128)], buf, sm).wait()

    def grp(g, carry):
        base = (c * GPC + g) * 8
        pltpu.sync_copy(gsrc.at[pl.ds(base, 8)], tsrc_v)
        pltpu.sync_copy(gdst.at[pl.ds(base, 8)], tdst_v)
        fire(tsrc_v.at[0], rows_a, sga)
        fire(tsrc_v.at[1], rows_b, sgb)
        for k in range(4):
            wait(rows_a, sga)
            pltpu.async_copy(rows_a, acc_t.at[tdst_v.at[2 * k]], ssa,
                             add=True)
            wait(rows_b, sgb)
            pltpu.async_copy(rows_b, acc_t.at[tdst_v.at[2 * k + 1]], ssb,
                             add=True)
            wait(rows_a, ssa)
            if k < 3:
                fire(tsrc_v.at[2 * k + 2], rows_a, sga)
            wait(rows_b, ssb)
            if k < 3:
                fire(tsrc_v.at[2 * k + 3], rows_b, sgb)
        return carry

    lax.fori_loop(g_lo, g_hi, grp, 0)
    plsc.subcore_barrier()
    pltpu.sync_copy(acc_t.at[pl.ds(s * 624, 624)],
                    z2.at[pl.ds(c * NT + s * 624, 624)])

    @pl.when(s == 15)
    def _otail():
        pltpu.sync_copy(acc_t.at[pl.ds(9984, 16)],
                        z2.at[pl.ds(c * NT + 9984, 16)])


def _scatter_obj_body(y_obj, osrc, odst, zz, z_obj,
                      osrc_v, odst_v, rows_a, rows_b, acc_o,
                      sga, sgb, ssa, ssb):
    c = lax.axis_index("c")
    s = lax.axis_index("s")
    pltpu.sync_copy(zz.at[pl.ds(0, 128)], acc_o.at[pl.ds(s * 128, 128)])
    plsc.subcore_barrier()
    pltpu.sync_copy(osrc.at[pl.ds(c * (CO // 2) + s * 16, 16)], osrc_v)
    pltpu.sync_copy(odst.at[pl.ds(c * (CO // 2) + s * 16, 16)], odst_v)

    def fire(j, buf, sm):
        pltpu.async_copy(y_obj.at[osrc_v.at[j]], buf, sm)

    def wait(buf, sm):
        pltpu.make_async_copy(y_obj.at[pl.ds(0, 128)], buf, sm).wait()

    fire(0, rows_a, sga)
    fire(1, rows_b, sgb)

    def ob(k, carry):
        j0 = 2 * k
        wait(rows_a, sga)
        pltpu.async_copy(rows_a, acc_o.at[odst_v.at[j0]], ssa, add=True)
        wait(rows_b, sgb)
        pltpu.async_copy(rows_b, acc_o.at[odst_v.at[j0 + 1]], ssb, add=True)
        wait(rows_a, ssa)

        @pl.when(2 * k + 2 < 16)
        def _ra():
            fire(j0 + 2, rows_a, sga)

        wait(rows_b, ssb)

        @pl.when(2 * k + 3 < 16)
        def _rb():
            fire(j0 + 3, rows_b, sgb)

        return carry

    lax.fori_loop(0, 8, ob, 0)
    plsc.subcore_barrier()
    pltpu.sync_copy(acc_o.at[pl.ds(s * 128, 128)],
                    z_obj.at[pl.ds(c * NO + s * 128, 128)])


@functools.cache
def _sc_kernels():
    mesh = plsc.VectorSubcoreMesh(core_axis_name="c", subcore_axis_name="s",
                                  num_cores=2, num_subcores=16)
    deg = pl.kernel(
        _deg_body,
        out_type=(jax.ShapeDtypeStruct((2 * NT, 16), F32),
                  jax.ShapeDtypeStruct((2 * NO, 16), F32)),
        mesh=mesh,
        scratch_types=[
            pltpu.VMEM((TPT, 128), jnp.int32),
            pltpu.VMEM((16, 128), jnp.int32),
            pltpu.VMEM((128, 16), F32),
            pltpu.VMEM_SHARED((NT, 16), F32),
            pltpu.VMEM_SHARED((NO, 16), F32),
        ],
    )
    scat_g = pl.kernel(
        _scatter_graph_body,
        out_type=jax.ShapeDtypeStruct((2 * NT, D), F32),
        mesh=mesh,
        scratch_types=[
            pltpu.VMEM((8, 128), jnp.int32),
            pltpu.VMEM((8, 128), jnp.int32),
            pltpu.VMEM((128, D), F32),
            pltpu.VMEM((128, D), F32),
            pltpu.VMEM_SHARED((NT + 8, D), F32),
            pltpu.SemaphoreType.DMA,
            pltpu.SemaphoreType.DMA,
            pltpu.SemaphoreType.DMA,
            pltpu.SemaphoreType.DMA,
        ],
    )
    scat_o = pl.kernel(
        _scatter_obj_body,
        out_type=jax.ShapeDtypeStruct((2 * NO, D), F32),
        mesh=mesh,
        scratch_types=[
            pltpu.VMEM((16, 128), jnp.int32),
            pltpu.VMEM((16, 128), jnp.int32),
            pltpu.VMEM((128, D), F32),
            pltpu.VMEM((128, D), F32),
            pltpu.VMEM_SHARED((NO, D), F32),
            pltpu.SemaphoreType.DMA,
            pltpu.SemaphoreType.DMA,
            pltpu.SemaphoreType.DMA,
            pltpu.SemaphoreType.DMA,
        ],
    )
    return deg, scat_g, scat_o


TM = 2000
NTB = NT // TM


def _dinv_txt(deg_blk):
    return lax.rsqrt(deg_blk[:, 0:1] + 1.0)


def _k1t_first_body(we, w_syn, w_sem, deg_t3, y_txt3):
    x = we[...]
    y_txt3[0, :, :] = _dinv_txt(deg_t3[0]) * jnp.dot(
        x, w_syn[...], preferred_element_type=F32)
    y_txt3[1, :, :] = _dinv_txt(deg_t3[1]) * jnp.dot(
        x, w_sem[...], preferred_element_type=F32)


def _k1t_first(we, w_syn, w_sem, deg_t3):
    return pl.pallas_call(
        _k1t_first_body,
        grid=(NTB,),
        in_specs=[
            pl.BlockSpec((TM, 300), lambda i: (i, 0)),
            pl.BlockSpec((300, D), lambda i: (0, 0)),
            pl.BlockSpec((300, D), lambda i: (0, 0)),
            pl.BlockSpec((2, TM, 16), lambda i: (0, i, 0)),
        ],
        out_specs=pl.BlockSpec((2, TM, D), lambda i: (0, i, 0)),
        out_shape=jax.ShapeDtypeStruct((2, NT, D), F32),
    )(we, w_syn, w_sem, deg_t3)


def _k1t_mid_body(z_syn, y_syn_prev, ctx_in, b_syn_prev, w_syn, w_sem,
                  deg_t3, y_txt3):
    dv_syn = _dinv_txt(deg_t3[0])
    x_syn = dv_syn * (z_syn[0] + z_syn[1] + y_syn_prev[0]) + b_syn_prev[...]
    y_txt3[0, :, :] = dv_syn * jnp.dot(x_syn, w_syn[...],
                                       preferred_element_type=F32)
    y_txt3[1, :, :] = _dinv_txt(deg_t3[1]) * jnp.dot(
        ctx_in[...], w_sem[...], preferred_element_type=F32)


def _k1t_mid(z_txt3, y_txt3_prev, ctx_in, b_syn_prev, w_syn, w_sem, deg_t3):
    return pl.pallas_call(
        _k1t_mid_body,
        grid=(NTB,),
        in_specs=[
            pl.BlockSpec((2, TM, D), lambda i: (0, i, 0)),
            pl.BlockSpec((1, TM, D), lambda i: (0, i, 0)),
            pl.BlockSpec((TM, D), lambda i: (i, 0)),
            pl.BlockSpec((1, D), lambda i: (0, 0)),
            pl.BlockSpec((D, D), lambda i: (0, 0)),
            pl.BlockSpec((D, D), lambda i: (0, 0)),
            pl.BlockSpec((2, TM, 16), lambda i: (0, i, 0)),
        ],
        out_specs=pl.BlockSpec((2, TM, D), lambda i: (0, i, 0)),
        out_shape=jax.ShapeDtypeStruct((2, NT, D), F32),
    )(z_txt3, y_txt3_prev, ctx_in, b_syn_prev, w_syn, w_sem, deg_t3)


def _k1o_body(obj_in, w_obj, deg_o, y_obj):
    dv_obj = lax.rsqrt(deg_o[0:NO, 0:1] + deg_o[NO:2 * NO, 0:1] + 1.0)
    y_obj[...] = dv_obj * jnp.dot(obj_in[...], w_obj[...],
                                  preferred_element_type=F32)


def _k1o(obj_in, w_obj, deg_o):
    return pl.pallas_call(
        _k1o_body,
        out_shape=jax.ShapeDtypeStruct((NO, D), F32),
    )(obj_in, w_obj, deg_o)


def _k3_body(z_ctx, y_ctx, deg_c, b_sem, z_obj, y_obj, deg_o, b_obj, wk, wq,
             ctx_out, obj_out, obj_gcn_s, kk_s, acc_s, m_s, s_s):
    i = pl.program_id(0)

    @pl.when(i == 0)
    def _init():
        dv_o = lax.rsqrt(deg_o[0:NO, 0:1] + deg_o[NO:2 * NO, 0:1] + 1.0)
        og = dv_o * (z_obj[0:NO, :] + z_obj[NO:2 * NO, :] + y_obj[...]) \
            + b_obj[...]
        obj_gcn_s[...] = og
        kk_s[...] = jnp.dot(og, wk[...], preferred_element_type=F32) \
            * (1.0 / jnp.sqrt(jnp.float32(D)))
        m_s[...] = jnp.full((NO, D), -1e30, F32)
        s_s[...] = jnp.zeros((NO, D), F32)
        acc_s[...] = jnp.zeros((NO, D), F32)

    dv_c = _dinv_txt(deg_c[0])
    ctx_gcn = dv_c * (z_ctx[0] + z_ctx[1] + y_ctx[0]) + b_sem[...]
    qq = jnp.dot(ctx_gcn, wq[...], preferred_element_type=F32)
    logits = lax.dot_general(kk_s[...], qq, (((1,), (1,)), ((), ())),
                             preferred_element_type=F32)
    m_run = m_s[...]
    m_step = jnp.max(logits, axis=1, keepdims=True)
    m_new = jnp.maximum(m_run, jnp.broadcast_to(m_step, (NO, D)))
    e1 = jnp.exp(logits - m_new[:, 0:1])
    big_m = jnp.max(m_new)
    v = jnp.exp(m_new[:, 0:1] - big_m)
    g = e1 * v
    s_c = jnp.sum(g, axis=0, keepdims=True)
    ctx_out[...] = lax.dot_general(g / s_c, obj_gcn_s[...],
                                   (((0,), (0,)), ((), ())),
                                   preferred_element_type=F32)
    alpha = jnp.exp(m_run - m_new)
    r_step = jnp.sum(e1, axis=1, keepdims=True)
    s_s[...] = s_s[...] * alpha + jnp.broadcast_to(r_step, (NO, D))
    acc_s[...] = acc_s[...] * alpha + jnp.dot(e1, ctx_gcn,
                                              preferred_element_type=F32)
    m_s[...] = m_new

    @pl.when(i == NSTEP - 1)
    def _fin():
        obj_out[...] = acc_s[...] / s_s[...]


def _k3(z_txt3, y_txt3, deg_t3, b_sem, z_obj, y_obj, deg_o, b_obj, wk, wq):
    return pl.pallas_call(
        _k3_body,
        grid=(NSTEP,),
        in_specs=[
            pl.BlockSpec((2, TQ, D), lambda i: (0, i, 0)),
            pl.BlockSpec((1, TQ, D), lambda i: (1, i, 0)),
            pl.BlockSpec((1, TQ, 16), lambda i: (1, i, 0)),
            pl.BlockSpec((1, D), lambda i: (0, 0)),
            pl.BlockSpec((2 * NO, D), lambda i: (0, 0)),
            pl.BlockSpec((NO, D), lambda i: (0, 0)),
            pl.BlockSpec((2 * NO, 16), lambda i: (0, 0)),
            pl.BlockSpec((1, D), lambda i: (0, 0)),
            pl.BlockSpec((D, D), lambda i: (0, 0)),
            pl.BlockSpec((D, D), lambda i: (0, 0)),
        ],
        out_specs=(pl.BlockSpec((TQ, D), lambda i: (i, 0)),
                   pl.BlockSpec((NO, D), lambda i: (0, 0))),
        out_shape=(jax.ShapeDtypeStruct((NT, D), F32),
                   jax.ShapeDtypeStruct((NO, D), F32)),
        scratch_shapes=[
            pltpu.VMEM((NO, D), F32),
            pltpu.VMEM((NO, D), F32),
            pltpu.VMEM((NO, D), F32),
            pltpu.VMEM((NO, D), F32),
            pltpu.VMEM((NO, D), F32),
        ],
    )(z_txt3, y_txt3, deg_t3, b_sem, z_obj, y_obj, deg_o, b_obj, wk, wq)


def _k4_body(z_syn, y_syn, deg_t, b_syn, ctx_fin, obj_fin, tb, ob,
             wf1, bf1, wf2, bf2, out):
    dv_syn = _dinv_txt(deg_t[0])
    syn_fin = dv_syn * (z_syn[0] + z_syn[1] + y_syn[0]) + b_syn[...]
    iot_t = lax.broadcasted_iota(jnp.int32, (NB, NT), 0)
    oh_t = (tb[...] == iot_t).astype(F32)
    ct = jnp.maximum(jnp.sum(oh_t, axis=1, keepdims=True), 1.0)
    iot_o = lax.broadcasted_iota(jnp.int32, (NB, NO), 0)
    oh_o = (ob[...] == iot_o).astype(F32)
    co = jnp.maximum(jnp.sum(oh_o, axis=1, keepdims=True), 1.0)
    syn_p = jnp.dot(oh_t, syn_fin, preferred_element_type=F32) / ct
    ctx_p = jnp.dot(oh_t, ctx_fin[...], preferred_element_type=F32) / ct
    obj_p = jnp.dot(oh_o, obj_fin[...], preferred_element_type=F32) / co
    fused = jnp.concatenate([syn_p, obj_p, ctx_p], axis=1)
    h = jnp.dot(fused, wf1[...], preferred_element_type=F32) + bf1[...]
    lg = jnp.dot(h, wf2[...], preferred_element_type=F32) + bf2[...]
    mm = jnp.max(lg, axis=1, keepdims=True)
    out[...] = (lg - mm) - jnp.log(
        jnp.sum(jnp.exp(lg - mm), axis=1, keepdims=True))


def _k4(z_txt, y_txt, deg_t, b_syn, ctx_fin, obj_fin, tb, ob,
        wf1, bf1, wf2, bf2):
    hid = wf1.shape[1]
    return pl.pallas_call(
        _k4_body,
        grid=(1,),
        in_specs=[
            pl.BlockSpec((2, NT, D), lambda i: (0, 0, 0)),
            pl.BlockSpec((1, NT, D), lambda i: (0, 0, 0)),
            pl.BlockSpec((1, NT, 16), lambda i: (0, 0, 0)),
            pl.BlockSpec((1, D), lambda i: (0, 0)),
            pl.BlockSpec((NT, D), lambda i: (0, 0)),
            pl.BlockSpec((NO, D), lambda i: (0, 0)),
            pl.BlockSpec((1, NT), lambda i: (0, 0)),
            pl.BlockSpec((1, NO), lambda i: (0, 0)),
            pl.BlockSpec((3 * D, hid), lambda i: (0, 0)),
            pl.BlockSpec((1, hid), lambda i: (0, 0)),
            pl.BlockSpec((hid, NA), lambda i: (0, 0)),
            pl.BlockSpec((1, NA), lambda i: (0, 0)),
        ],
        out_specs=pl.BlockSpec((NB, NA), lambda i: (0, 0)),
        out_shape=jax.ShapeDtypeStruct((NB, NA), F32),
    )(z_txt, y_txt, deg_t, b_syn, ctx_fin, obj_fin, tb, ob,
      wf1, bf1, wf2, bf2)


def kernel(word_embeddings, objects, syn_edge_index, ctx_edge_index,
           obj_edge_index, txt_batch, obj_batch, params):
    p = params
    syn_src = syn_edge_index[0].astype(jnp.int32)
    syn_dst = syn_edge_index[1].astype(jnp.int32)
    ctx_src = ctx_edge_index[0].astype(jnp.int32)
    ctx_dst = ctx_edge_index[1].astype(jnp.int32)
    tdst = jnp.pad(jnp.stack([syn_dst, ctx_dst]).reshape(2, CT, 128),
                   ((0, 0), (0, CTP - CT), (0, 0)))
    ssrc = jnp.pad(syn_src.reshape(CT, 128), ((0, CTG - CT), (0, 0)))
    sdst = jnp.pad(syn_dst.reshape(CT, 128), ((0, CTG - CT), (0, 0)),
                   constant_values=NT)
    csrc = jnp.pad(ctx_src.reshape(CT, 128), ((0, CTG - CT), (0, 0))) + NT
    cdst = jnp.pad(ctx_dst.reshape(CT, 128), ((0, CTG - CT), (0, 0)),
                   constant_values=NT)
    osrc = obj_edge_index[0].astype(jnp.int32).reshape(CO, 128)
    odst = obj_edge_index[1].astype(jnp.int32).reshape(CO, 128)
    ones16 = jnp.ones((128, 16), F32)
    zz16 = jnp.zeros((624, 16), F32)
    zz = jnp.zeros((624, D), F32)
    tb = txt_batch.astype(jnp.int32).reshape(1, NT)
    ob = obj_batch.astype(jnp.int32).reshape(1, NO)

    sc_deg, sc_scat_g, sc_scat_o = _sc_kernels()
    deg_t, deg_o = sc_deg(tdst, odst, ones16, zz16)
    deg_t3 = deg_t.reshape(2, NT, 16)

    ctx_in = None
    obj_in = None
    y_txt3 = y_obj = z_syn2 = z_ctx2 = z_obj = None
    for l in range(3):
        if l == 0:
            y_txt3 = _k1t_first(word_embeddings, p['W_syn_0'], p['W_sem_0'],
                                deg_t3)
            y_obj = _k1o(objects, p['W_obj_0'], deg_o)
        else:
            y_txt3 = _k1t_mid(z_syn2, y_txt3, ctx_in,
                              p['b_syn_%d' % (l - 1)].reshape(1, D),
                              p['W_syn_%d' % l], p['W_sem_%d' % l], deg_t3)
            y_obj = _k1o(obj_in, p['W_obj_%d' % l], deg_o)
        y_flat = y_txt3.reshape(2 * NT, D)
        z_ctx2 = sc_scat_g(y_flat, csrc, cdst, zz).reshape(2, NT, D)
        z_obj = sc_scat_o(y_obj, osrc, odst, zz)
        z_syn2 = sc_scat_g(y_flat, ssrc, sdst, zz).reshape(2, NT, D)
        ctx_in, obj_in = _k3(
            z_ctx2, y_txt3, deg_t3, p['b_sem_%d' % l].reshape(1, D),
            z_obj, y_obj, deg_o, p['b_obj_%d' % l].reshape(1, D),
            p['Wk_%d' % l], p['Wq_%d' % l])

    hid = p['Wf1'].shape[1]
    return _k4(z_syn2, y_txt3, deg_t3, p['b_syn_2'].reshape(1, D),
               ctx_in, obj_in, tb, ob,
               p['Wf1'], p['bf1'].reshape(1, hid),
               p['Wf2'], p['bf2'].reshape(1, NA))


# speedup vs baseline: 1.3494x; 1.3494x over previous
import functools
import jax
import jax.numpy as jnp
from jax import lax
from jax.experimental import pallas as pl
from jax.experimental.pallas import tpu as pltpu
from jax.experimental.pallas import tpu_sc as plsc

NT = 10000
NO = 2048
ET = 160000
EO = 65536
NB = 64
NA = 3129
D = 128
CT = ET // 128
CTP = 1256
CT64 = ET // 64
CTP64 = 2504
NG64 = CTP64 // 8
CO = EO // 128
TPT = 88
TQ = 1000
NSTEP = NT // TQ

F32 = jnp.float32


def _deg_body(tdst, odst, ones16, zz16, deg_t, deg_o, tidx, oidx, ones_v,
              acc_t, acc_o):
    c = lax.axis_index("c")
    s = lax.axis_index("s")
    pltpu.sync_copy(zz16, acc_t.at[pl.ds(s * 624, 624)])
    pltpu.sync_copy(zz16.at[pl.ds(0, 128)], acc_o.at[pl.ds(s * 128, 128)])

    @pl.when(s == 15)
    def _ztail():
        pltpu.sync_copy(zz16.at[pl.ds(0, 16)], acc_t.at[pl.ds(9984, 16)])

    pltpu.sync_copy(ones16, ones_v)
    plsc.subcore_barrier()
    lo = s * CT // 16
    hi = (s + 1) * CT // 16
    lo8 = (lo // 8) * 8
    pltpu.sync_copy(tdst.at[c, pl.ds(lo8, TPT)], tidx)
    pltpu.sync_copy(odst.at[pl.ds(c * (CO // 2) + s * 16, 16)], oidx)

    def tb(j, carry):
        pltpu.sync_copy(ones_v, acc_t.at[tidx.at[j]], add=True)
        return carry

    lax.fori_loop(lo - lo8, hi - lo8, tb, 0)

    def ob(j, carry):
        pltpu.sync_copy(ones_v, acc_o.at[oidx.at[j]], add=True)
        return carry

    lax.fori_loop(0, 16, ob, 0)
    plsc.subcore_barrier()
    pltpu.sync_copy(acc_t.at[pl.ds(s * 624, 624)],
                    deg_t.at[pl.ds(c * NT + s * 624, 624)])
    pltpu.sync_copy(acc_o.at[pl.ds(s * 128, 128)],
                    deg_o.at[pl.ds(c * NO + s * 128, 128)])

    @pl.when(s == 15)
    def _otail():
        pltpu.sync_copy(acc_t.at[pl.ds(9984, 16)],
                        deg_t.at[pl.ds(c * NT + 9984, 16)])


def _scatter_txt_body(y_txt, tsrc, tdst, zz, z_txt,
                      tsrc_v, tdst_v, r0, r1, r2, r3, acc_t,
                      sg0, sg1, sg2, sg3, ss0, ss1, ss2, ss3):
    c = lax.axis_index("c")
    s = lax.axis_index("s")
    pltpu.sync_copy(zz, acc_t.at[pl.ds(s * 624, 624)])

    @pl.when(s == 15)
    def _ztail():
        pltpu.sync_copy(zz.at[pl.ds(0, 24)], acc_t.at[pl.ds(9984, 24)])

    plsc.subcore_barrier()
    g_lo = s * NG64 // 16
    g_hi = (s + 1) * NG64 // 16
    bufs = [(r0, sg0, ss0), (r1, sg1, ss1), (r2, sg2, ss2), (r3, sg3, ss3)]

    def grp(g, carry):
        pltpu.sync_copy(tsrc.at[c, pl.ds(g * 8, 8)], tsrc_v)
        pltpu.sync_copy(tdst.at[c, pl.ds(g * 8, 8)], tdst_v)
        for t in range(10):
            if t < 8:
                buf, sg, ss = bufs[t % 4]
                if t >= 4:
                    pltpu.make_async_copy(y_txt.at[pl.ds(0, 64)], buf,
                                          ss).wait()
                pltpu.async_copy(y_txt.at[tsrc_v.at[t]], buf, sg)
            if 2 <= t:
                j = t - 2
                buf, sg, ss = bufs[j % 4]
                pltpu.make_async_copy(y_txt.at[pl.ds(0, 64)], buf, sg).wait()
                pltpu.async_copy(buf, acc_t.at[tdst_v.at[j]], ss, add=True)
        for b in range(4):
            pltpu.make_async_copy(y_txt.at[pl.ds(0, 64)], bufs[b][0],
                                  bufs[b][2]).wait()
        return carry

    lax.fori_loop(g_lo, g_hi, grp, 0)
    plsc.subcore_barrier()
    pltpu.sync_copy(acc_t.at[pl.ds(s * 624, 624)],
                    z_txt.at[pl.ds(c * NT + s * 624, 624)])

    @pl.when(s == 15)
    def _otail():
        pltpu.sync_copy(acc_t.at[pl.ds(9984, 16)],
                        z_txt.at[pl.ds(c * NT + 9984, 16)])


def _scatter_obj_body(y_obj, osrc, odst, zz, z_obj,
                      osrc_v, odst_v, rows_a, rows_b, acc_o,
                      sga, sgb, ssa, ssb):
    c = lax.axis_index("c")
    s = lax.axis_index("s")
    pltpu.sync_copy(zz.at[pl.ds(0, 128)], acc_o.at[pl.ds(s * 128, 128)])
    plsc.subcore_barrier()
    pltpu.sync_copy(osrc.at[pl.ds(c * (CO // 2) + s * 16, 16)], osrc_v)
    pltpu.sync_copy(odst.at[pl.ds(c * (CO // 2) + s * 16, 16)], odst_v)

    def fire(j, buf, sm):
        pltpu.async_copy(y_obj.at[osrc_v.at[j]], buf, sm)

    def wait(buf, sm):
        pltpu.make_async_copy(y_obj.at[pl.ds(0, 128)], buf, sm).wait()

    fire(0, rows_a, sga)
    fire(1, rows_b, sgb)

    def ob(k, carry):
        j0 = 2 * k
        wait(rows_a, sga)
        pltpu.async_copy(rows_a, acc_o.at[odst_v.at[j0]], ssa, add=True)
        wait(rows_b, sgb)
        pltpu.async_copy(rows_b, acc_o.at[odst_v.at[j0 + 1]], ssb, add=True)
        wait(rows_a, ssa)

        @pl.when(2 * k + 2 < 16)
        def _ra():
            fire(j0 + 2, rows_a, sga)

        wait(rows_b, ssb)

        @pl.when(2 * k + 3 < 16)
        def _rb():
            fire(j0 + 3, rows_b, sgb)

        return carry

    lax.fori_loop(0, 8, ob, 0)
    plsc.subcore_barrier()
    pltpu.sync_copy(acc_o.at[pl.ds(s * 128, 128)],
                    z_obj.at[pl.ds(c * NO + s * 128, 128)])


@functools.cache
def _sc_kernels():
    mesh = plsc.VectorSubcoreMesh(core_axis_name="c", subcore_axis_name="s",
                                  num_cores=2, num_subcores=16)
    deg = pl.kernel(
        _deg_body,
        out_type=(jax.ShapeDtypeStruct((2 * NT, 16), F32),
                  jax.ShapeDtypeStruct((2 * NO, 16), F32)),
        mesh=mesh,
        scratch_types=[
            pltpu.VMEM((TPT, 128), jnp.int32),
            pltpu.VMEM((16, 128), jnp.int32),
            pltpu.VMEM((128, 16), F32),
            pltpu.VMEM_SHARED((NT, 16), F32),
            pltpu.VMEM_SHARED((NO, 16), F32),
        ],
    )
    scat_t = pl.kernel(
        _scatter_txt_body,
        out_type=jax.ShapeDtypeStruct((2 * NT, D), F32),
        mesh=mesh,
        scratch_types=[
            pltpu.VMEM((8, 64), jnp.int32),
            pltpu.VMEM((8, 64), jnp.int32),
            pltpu.VMEM((64, D), F32),
            pltpu.VMEM((64, D), F32),
            pltpu.VMEM((64, D), F32),
            pltpu.VMEM((64, D), F32),
            pltpu.VMEM_SHARED((NT + 8, D), F32),
        ] + [pltpu.SemaphoreType.DMA] * 8,
    )
    scat_o = pl.kernel(
        _scatter_obj_body,
        out_type=jax.ShapeDtypeStruct((2 * NO, D), F32),
        mesh=mesh,
        scratch_types=[
            pltpu.VMEM((16, 128), jnp.int32),
            pltpu.VMEM((16, 128), jnp.int32),
            pltpu.VMEM((128, D), F32),
            pltpu.VMEM((128, D), F32),
            pltpu.VMEM_SHARED((NO, D), F32),
            pltpu.SemaphoreType.DMA,
            pltpu.SemaphoreType.DMA,
            pltpu.SemaphoreType.DMA,
            pltpu.SemaphoreType.DMA,
        ],
    )
    return deg, scat_t, scat_o


TM = 2000
NTB = NT // TM


def _dinv_txt(deg_blk):
    return lax.rsqrt(deg_blk[:, 0:1] + 1.0)


def _k1t_first_body(we, w_syn, w_sem, deg_t3, y_txt3):
    x = we[...]
    y_txt3[0, :, :] = _dinv_txt(deg_t3[0]) * jnp.dot(
        x, w_syn[...], preferred_element_type=F32)
    y_txt3[1, :, :] = _dinv_txt(deg_t3[1]) * jnp.dot(
        x, w_sem[...], preferred_element_type=F32)


def _k1t_first(we, w_syn, w_sem, deg_t3):
    return pl.pallas_call(
        _k1t_first_body,
        grid=(NTB,),
        in_specs=[
            pl.BlockSpec((TM, 300), lambda i: (i, 0)),
            pl.BlockSpec((300, D), lambda i: (0, 0)),
            pl.BlockSpec((300, D), lambda i: (0, 0)),
            pl.BlockSpec((2, TM, 16), lambda i: (0, i, 0)),
        ],
        out_specs=pl.BlockSpec((2, TM, D), lambda i: (0, i, 0)),
        out_shape=jax.ShapeDtypeStruct((2, NT, D), F32),
    )(we, w_syn, w_sem, deg_t3)


def _k1t_mid_body(z_syn, y_syn_prev, ctx_in, b_syn_prev, w_syn, w_sem,
                  deg_t3, y_txt3):
    dv_syn = _dinv_txt(deg_t3[0])
    x_syn = dv_syn * (z_syn[0] + y_syn_prev[0]) + b_syn_prev[...]
    y_txt3[0, :, :] = dv_syn * jnp.dot(x_syn, w_syn[...],
                                       preferred_element_type=F32)
    y_txt3[1, :, :] = _dinv_txt(deg_t3[1]) * jnp.dot(
        ctx_in[...], w_sem[...], preferred_element_type=F32)


def _k1t_mid(z_txt3, y_txt3_prev, ctx_in, b_syn_prev, w_syn, w_sem, deg_t3):
    return pl.pallas_call(
        _k1t_mid_body,
        grid=(NTB,),
        in_specs=[
            pl.BlockSpec((1, TM, D), lambda i: (0, i, 0)),
            pl.BlockSpec((1, TM, D), lambda i: (0, i, 0)),
            pl.BlockSpec((TM, D), lambda i: (i, 0)),
            pl.BlockSpec((1, D), lambda i: (0, 0)),
            pl.BlockSpec((D, D), lambda i: (0, 0)),
            pl.BlockSpec((D, D), lambda i: (0, 0)),
            pl.BlockSpec((2, TM, 16), lambda i: (0, i, 0)),
        ],
        out_specs=pl.BlockSpec((2, TM, D), lambda i: (0, i, 0)),
        out_shape=jax.ShapeDtypeStruct((2, NT, D), F32),
    )(z_txt3, y_txt3_prev, ctx_in, b_syn_prev, w_syn, w_sem, deg_t3)


def _k1o_body(obj_in, w_obj, deg_o, y_obj):
    dv_obj = lax.rsqrt(deg_o[0:NO, 0:1] + deg_o[NO:2 * NO, 0:1] + 1.0)
    y_obj[...] = dv_obj * jnp.dot(obj_in[...], w_obj[...],
                                  preferred_element_type=F32)


def _k1o(obj_in, w_obj, deg_o):
    return pl.pallas_call(
        _k1o_body,
        out_shape=jax.ShapeDtypeStruct((NO, D), F32),
    )(obj_in, w_obj, deg_o)


def _k3_body(z_ctx, y_ctx, deg_c, b_sem, z_obj, y_obj, deg_o, b_obj, wk, wq,
             ctx_out, obj_out, obj_gcn_s, kk_s, acc_s, m_s, s_s):
    i = pl.program_id(0)

    @pl.when(i == 0)
    def _init():
        dv_o = lax.rsqrt(deg_o[0:NO, 0:1] + deg_o[NO:2 * NO, 0:1] + 1.0)
        og = dv_o * (z_obj[0:NO, :] + z_obj[NO:2 * NO, :] + y_obj[...]) \
            + b_obj[...]
        obj_gcn_s[...] = og
        kk_s[...] = jnp.dot(og, wk[...], preferred_element_type=F32) \
            * (1.0 / jnp.sqrt(jnp.float32(D)))
        m_s[...] = jnp.full((NO, D), -1e30, F32)
        s_s[...] = jnp.zeros((NO, D), F32)
        acc_s[...] = jnp.zeros((NO, D), F32)

    dv_c = _dinv_txt(deg_c[0])
    ctx_gcn = dv_c * (z_ctx[0] + y_ctx[0]) + b_sem[...]
    qq = jnp.dot(ctx_gcn, wq[...], preferred_element_type=F32)
    logits = lax.dot_general(kk_s[...], qq, (((1,), (1,)), ((), ())),
                             preferred_element_type=F32)
    m_run = m_s[...]
    m_step = jnp.max(logits, axis=1, keepdims=True)
    m_new = jnp.maximum(m_run, jnp.broadcast_to(m_step, (NO, D)))
    e1 = jnp.exp(logits - m_new[:, 0:1])
    big_m = jnp.max(m_new)
    v = jnp.exp(m_new[:, 0:1] - big_m)
    g = e1 * v
    s_c = jnp.sum(g, axis=0, keepdims=True)
    ctx_out[...] = lax.dot_general(g / s_c, obj_gcn_s[...],
                                   (((0,), (0,)), ((), ())),
                                   preferred_element_type=F32)
    alpha = jnp.exp(m_run - m_new)
    r_step = jnp.sum(e1, axis=1, keepdims=True)
    s_s[...] = s_s[...] * alpha + jnp.broadcast_to(r_step, (NO, D))
    acc_s[...] = acc_s[...] * alpha + jnp.dot(e1, ctx_gcn,
                                              preferred_element_type=F32)
    m_s[...] = m_new

    @pl.when(i == NSTEP - 1)
    def _fin():
        obj_out[...] = acc_s[...] / s_s[...]


def _k3(z_txt3, y_txt3, deg_t3, b_sem, z_obj, y_obj, deg_o, b_obj, wk, wq):
    return pl.pallas_call(
        _k3_body,
        grid=(NSTEP,),
        in_specs=[
            pl.BlockSpec((1, TQ, D), lambda i: (1, i, 0)),
            pl.BlockSpec((1, TQ, D), lambda i: (1, i, 0)),
            pl.BlockSpec((1, TQ, 16), lambda i: (1, i, 0)),
            pl.BlockSpec((1, D), lambda i: (0, 0)),
            pl.BlockSpec((2 * NO, D), lambda i: (0, 0)),
            pl.BlockSpec((NO, D), lambda i: (0, 0)),
            pl.BlockSpec((2 * NO, 16), lambda i: (0, 0)),
            pl.BlockSpec((1, D), lambda i: (0, 0)),
            pl.BlockSpec((D, D), lambda i: (0, 0)),
            pl.BlockSpec((D, D), lambda i: (0, 0)),
        ],
        out_specs=(pl.BlockSpec((TQ, D), lambda i: (i, 0)),
                   pl.BlockSpec((NO, D), lambda i: (0, 0))),
        out_shape=(jax.ShapeDtypeStruct((NT, D), F32),
                   jax.ShapeDtypeStruct((NO, D), F32)),
        scratch_shapes=[
            pltpu.VMEM((NO, D), F32),
            pltpu.VMEM((NO, D), F32),
            pltpu.VMEM((NO, D), F32),
            pltpu.VMEM((NO, D), F32),
            pltpu.VMEM((NO, D), F32),
        ],
    )(z_txt3, y_txt3, deg_t3, b_sem, z_obj, y_obj, deg_o, b_obj, wk, wq)


def _k4_body(z_syn, y_syn, deg_t, b_syn, ctx_fin, obj_fin, tb, ob,
             wf1, bf1, wf2, bf2, out):
    dv_syn = _dinv_txt(deg_t[0])
    syn_fin = dv_syn * (z_syn[0] + y_syn[0]) + b_syn[...]
    iot_t = lax.broadcasted_iota(jnp.int32, (NB, NT), 0)
    oh_t = (tb[...] == iot_t).astype(F32)
    ct = jnp.maximum(jnp.sum(oh_t, axis=1, keepdims=True), 1.0)
    iot_o = lax.broadcasted_iota(jnp.int32, (NB, NO), 0)
    oh_o = (ob[...] == iot_o).astype(F32)
    co = jnp.maximum(jnp.sum(oh_o, axis=1, keepdims=True), 1.0)
    syn_p = jnp.dot(oh_t, syn_fin, preferred_element_type=F32) / ct
    ctx_p = jnp.dot(oh_t, ctx_fin[...], preferred_element_type=F32) / ct
    obj_p = jnp.dot(oh_o, obj_fin[...], preferred_element_type=F32) / co
    fused = jnp.concatenate([syn_p, obj_p, ctx_p], axis=1)
    h = jnp.dot(fused, wf1[...], preferred_element_type=F32) + bf1[...]
    lg = jnp.dot(h, wf2[...], preferred_element_type=F32) + bf2[...]
    mm = jnp.max(lg, axis=1, keepdims=True)
    out[...] = (lg - mm) - jnp.log(
        jnp.sum(jnp.exp(lg - mm), axis=1, keepdims=True))


def _k4(z_txt, y_txt, deg_t, b_syn, ctx_fin, obj_fin, tb, ob,
        wf1, bf1, wf2, bf2):
    hid = wf1.shape[1]
    return pl.pallas_call(
        _k4_body,
        grid=(1,),
        in_specs=[
            pl.BlockSpec((1, NT, D), lambda i: (0, 0, 0)),
            pl.BlockSpec((1, NT, D), lambda i: (0, 0, 0)),
            pl.BlockSpec((1, NT, 16), lambda i: (0, 0, 0)),
            pl.BlockSpec((1, D), lambda i: (0, 0)),
            pl.BlockSpec((NT, D), lambda i: (0, 0)),
            pl.BlockSpec((NO, D), lambda i: (0, 0)),
            pl.BlockSpec((1, NT), lambda i: (0, 0)),
            pl.BlockSpec((1, NO), lambda i: (0, 0)),
            pl.BlockSpec((3 * D, hid), lambda i: (0, 0)),
            pl.BlockSpec((1, hid), lambda i: (0, 0)),
            pl.BlockSpec((hid, NA), lambda i: (0, 0)),
            pl.BlockSpec((1, NA), lambda i: (0, 0)),
        ],
        out_specs=pl.BlockSpec((NB, NA), lambda i: (0, 0)),
        out_shape=jax.ShapeDtypeStruct((NB, NA), F32),
    )(z_txt, y_txt, deg_t, b_syn, ctx_fin, obj_fin, tb, ob,
      wf1, bf1, wf2, bf2)


def kernel(word_embeddings, objects, syn_edge_index, ctx_edge_index,
           obj_edge_index, txt_batch, obj_batch, params):
    p = params
    syn_src = syn_edge_index[0].astype(jnp.int32)
    syn_dst = syn_edge_index[1].astype(jnp.int32)
    ctx_src = ctx_edge_index[0].astype(jnp.int32)
    ctx_dst = ctx_edge_index[1].astype(jnp.int32)
    tdst = jnp.pad(jnp.stack([syn_dst, ctx_dst]).reshape(2, CT, 128),
                   ((0, 0), (0, CTP - CT), (0, 0)))
    npad = CTP64 - CT64
    tsrc64 = jnp.stack([
        jnp.pad(syn_src.reshape(CT64, 64), ((0, npad), (0, 0))),
        jnp.pad(ctx_src.reshape(CT64, 64), ((0, npad), (0, 0))) + NT,
    ])
    tdst64 = jnp.stack([
        jnp.pad(syn_dst.reshape(CT64, 64), ((0, npad), (0, 0)),
                constant_values=NT),
        jnp.pad(ctx_dst.reshape(CT64, 64), ((0, npad), (0, 0)),
                constant_values=NT),
    ])
    osrc = obj_edge_index[0].astype(jnp.int32).reshape(CO, 128)
    odst = obj_edge_index[1].astype(jnp.int32).reshape(CO, 128)
    ones16 = jnp.ones((128, 16), F32)
    zz16 = jnp.zeros((624, 16), F32)
    zz = jnp.zeros((624, D), F32)
    tb = txt_batch.astype(jnp.int32).reshape(1, NT)
    ob = obj_batch.astype(jnp.int32).reshape(1, NO)

    sc_deg, sc_scat_t, sc_scat_o = _sc_kernels()
    deg_t, deg_o = sc_deg(tdst, odst, ones16, zz16)
    deg_t3 = deg_t.reshape(2, NT, 16)

    ctx_in = None
    obj_in = None
    y_txt3 = y_obj = z_txt3 = z_obj = None
    for l in range(3):
        if l == 0:
            y_txt3 = _k1t_first(word_embeddings, p['W_syn_0'], p['W_sem_0'],
                                deg_t3)
            y_obj = _k1o(objects, p['W_obj_0'], deg_o)
        else:
            y_txt3 = _k1t_mid(z_txt3, y_txt3, ctx_in,
                              p['b_syn_%d' % (l - 1)].reshape(1, D),
                              p['W_syn_%d' % l], p['W_sem_%d' % l], deg_t3)
            y_obj = _k1o(obj_in, p['W_obj_%d' % l], deg_o)
        y_flat = y_txt3.reshape(2 * NT, D)
        z_txt3 = sc_scat_t(y_flat, tsrc64, tdst64, zz).reshape(2, NT, D)
        z_obj = sc_scat_o(y_obj, osrc, odst, zz)
        ctx_in, obj_in = _k3(
            z_txt3, y_txt3, deg_t3, p['b_sem_%d' % l].reshape(1, D),
            z_obj, y_obj, deg_o, p['b_obj_%d' % l].reshape(1, D),
            p['Wk_%d' % l], p['Wq_%d' % l])

    hid = p['Wf1'].shape[1]
    return _k4(z_txt3, y_txt3, deg_t3, p['b_syn_2'].reshape(1, D),
               ctx_in, obj_in, tb, ob,
               p['Wf1'], p['bf1'].reshape(1, hid),
               p['Wf2'], p['bf2'].reshape(1, NA))


# speedup vs baseline: 1.3827x; 1.0247x over previous
import functools
import jax
import jax.numpy as jnp
from jax import lax
from jax.experimental import pallas as pl
from jax.experimental.pallas import tpu as pltpu
from jax.experimental.pallas import tpu_sc as plsc

NT = 10000
NO = 2048
ET = 160000
EO = 65536
NB = 64
NA = 3129
D = 128
CT = ET // 128
CTP = 1256
CT64 = ET // 64
CTG64 = 2512
NGG = 157
TRASH = 64
CO = EO // 128
TPT = 88
TQ = 1000
NSTEP = NT // TQ

F32 = jnp.float32


def _deg_body(tdst, odst, ones16, zz16, deg_t, deg_o, tidx, oidx, ones_v,
              acc_t, acc_o):
    c = lax.axis_index("c")
    s = lax.axis_index("s")
    pltpu.sync_copy(zz16, acc_t.at[pl.ds(s * 624, 624)])
    pltpu.sync_copy(zz16.at[pl.ds(0, 128)], acc_o.at[pl.ds(s * 128, 128)])

    @pl.when(s == 15)
    def _ztail():
        pltpu.sync_copy(zz16.at[pl.ds(0, 16)], acc_t.at[pl.ds(9984, 16)])

    pltpu.sync_copy(ones16, ones_v)
    plsc.subcore_barrier()
    lo = s * CT // 16
    hi = (s + 1) * CT // 16
    lo8 = (lo // 8) * 8
    pltpu.sync_copy(tdst.at[c, pl.ds(lo8, TPT)], tidx)
    pltpu.sync_copy(odst.at[pl.ds(c * (CO // 2) + s * 16, 16)], oidx)

    def tb(j, carry):
        pltpu.sync_copy(ones_v, acc_t.at[tidx.at[j]], add=True)
        return carry

    lax.fori_loop(lo - lo8, hi - lo8, tb, 0)

    def ob(j, carry):
        pltpu.sync_copy(ones_v, acc_o.at[oidx.at[j]], add=True)
        return carry

    lax.fori_loop(0, 16, ob, 0)
    plsc.subcore_barrier()
    pltpu.sync_copy(acc_t.at[pl.ds(s * 624, 624)],
                    deg_t.at[pl.ds(c * NT + s * 624, 624)])
    pltpu.sync_copy(acc_o.at[pl.ds(s * 128, 128)],
                    deg_o.at[pl.ds(c * NO + s * 128, 128)])

    @pl.when(s == 15)
    def _otail():
        pltpu.sync_copy(acc_t.at[pl.ds(9984, 16)],
                        deg_t.at[pl.ds(c * NT + 9984, 16)])


def _scatter_txt_body(y_txt, tsrc, tdst, zz, z2,
                      tsrc_v, tdst_v, r0, r1, r2, r3, acc_t,
                      sg0, sg1, sg2, sg3, ss0, ss1, ss2, ss3):
    c = lax.axis_index("c")
    s = lax.axis_index("s")
    pltpu.sync_copy(zz, acc_t.at[pl.ds(s * 624, 624)])

    @pl.when(s == 15)
    def _ztail():
        pltpu.sync_copy(zz.at[pl.ds(0, 16)], acc_t.at[pl.ds(9984, 16)])

    plsc.subcore_barrier()
    g_lo = s * NGG // 16
    g_hi = (s + 1) * NGG // 16
    bufs = [(r0, sg0, ss0), (r1, sg1, ss1), (r2, sg2, ss2), (r3, sg3, ss3)]

    def grp(g, carry):
        base = (c * NGG + g) * 8
        pltpu.sync_copy(tsrc.at[pl.ds(base, 8)], tsrc_v)
        pltpu.sync_copy(tdst.at[pl.ds(base, 8)], tdst_v)
        for t in range(10):
            if t < 8:
                buf, sg, ss = bufs[t % 4]
                if t >= 4:
                    pltpu.make_async_copy(y_txt.at[pl.ds(0, 64)], buf,
                                          ss).wait()
                pltpu.async_copy(y_txt.at[tsrc_v.at[t]], buf, sg)
            if 2 <= t:
                j = t - 2
                buf, sg, ss = bufs[j % 4]
                pltpu.make_async_copy(y_txt.at[pl.ds(0, 64)], buf, sg).wait()
                pltpu.async_copy(buf, acc_t.at[tdst_v.at[j]], ss, add=True)
        for b in range(4):
            pltpu.make_async_copy(y_txt.at[pl.ds(0, 64)], bufs[b][0],
                                  bufs[b][2]).wait()
        return carry

    lax.fori_loop(g_lo, g_hi, grp, 0)
    plsc.subcore_barrier()
    pltpu.sync_copy(acc_t.at[pl.ds(s * 624, 624)],
                    z2.at[pl.ds(c * NT + s * 624, 624)])

    @pl.when(s == 15)
    def _otail():
        pltpu.sync_copy(acc_t.at[pl.ds(9984, 16)],
                        z2.at[pl.ds(c * NT + 9984, 16)])


def _scatter_obj_body(y_obj, osrc, odst, zz, z_obj,
                      osrc_v, odst_v, rows_a, rows_b, acc_o,
                      sga, sgb, ssa, ssb):
    c = lax.axis_index("c")
    s = lax.axis_index("s")
    pltpu.sync_copy(zz.at[pl.ds(0, 128)], acc_o.at[pl.ds(s * 128, 128)])
    plsc.subcore_barrier()
    pltpu.sync_copy(osrc.at[pl.ds(c * (CO // 2) + s * 16, 16)], osrc_v)
    pltpu.sync_copy(odst.at[pl.ds(c * (CO // 2) + s * 16, 16)], odst_v)

    def fire(j, buf, sm):
        pltpu.async_copy(y_obj.at[osrc_v.at[j]], buf, sm)

    def wait(buf, sm):
        pltpu.make_async_copy(y_obj.at[pl.ds(0, 128)], buf, sm).wait()

    fire(0, rows_a, sga)
    fire(1, rows_b, sgb)

    def ob(k, carry):
        j0 = 2 * k
        wait(rows_a, sga)
        pltpu.async_copy(rows_a, acc_o.at[odst_v.at[j0]], ssa, add=True)
        wait(rows_b, sgb)
        pltpu.async_copy(rows_b, acc_o.at[odst_v.at[j0 + 1]], ssb, add=True)
        wait(rows_a, ssa)

        @pl.when(2 * k + 2 < 16)
        def _ra():
            fire(j0 + 2, rows_a, sga)

        wait(rows_b, ssb)

        @pl.when(2 * k + 3 < 16)
        def _rb():
            fire(j0 + 3, rows_b, sgb)

        return carry

    lax.fori_loop(0, 8, ob, 0)
    plsc.subcore_barrier()
    pltpu.sync_copy(acc_o.at[pl.ds(s * 128, 128)],
                    z_obj.at[pl.ds(c * NO + s * 128, 128)])


@functools.cache
def _sc_kernels():
    mesh = plsc.VectorSubcoreMesh(core_axis_name="c", subcore_axis_name="s",
                                  num_cores=2, num_subcores=16)
    deg = pl.kernel(
        _deg_body,
        out_type=(jax.ShapeDtypeStruct((2 * NT, 16), F32),
                  jax.ShapeDtypeStruct((2 * NO, 16), F32)),
        mesh=mesh,
        scratch_types=[
            pltpu.VMEM((TPT, 128), jnp.int32),
            pltpu.VMEM((16, 128), jnp.int32),
            pltpu.VMEM((128, 16), F32),
            pltpu.VMEM_SHARED((NT, 16), F32),
            pltpu.VMEM_SHARED((NO, 16), F32),
        ],
    )
    scat_t = pl.kernel(
        _scatter_txt_body,
        out_type=jax.ShapeDtypeStruct((2 * NT, D), F32),
        mesh=mesh,
        scratch_types=[
            pltpu.VMEM((8, 64), jnp.int32),
            pltpu.VMEM((8, 64), jnp.int32),
            pltpu.VMEM((64, D), F32),
            pltpu.VMEM((64, D), F32),
            pltpu.VMEM((64, D), F32),
            pltpu.VMEM((64, D), F32),
            pltpu.VMEM_SHARED((NT + TRASH, D), F32),
        ] + [pltpu.SemaphoreType.DMA] * 8,
    )
    scat_o = pl.kernel(
        _scatter_obj_body,
        out_type=jax.ShapeDtypeStruct((2 * NO, D), F32),
        mesh=mesh,
        scratch_types=[
            pltpu.VMEM((16, 128), jnp.int32),
            pltpu.VMEM((16, 128), jnp.int32),
            pltpu.VMEM((128, D), F32),
            pltpu.VMEM((128, D), F32),
            pltpu.VMEM_SHARED((NO, D), F32),
            pltpu.SemaphoreType.DMA,
            pltpu.SemaphoreType.DMA,
            pltpu.SemaphoreType.DMA,
            pltpu.SemaphoreType.DMA,
        ],
    )
    return deg, scat_t, scat_o


TM = 2000
NTB = NT // TM


def _dinv_txt(deg_blk):
    return lax.rsqrt(deg_blk[:, 0:1] + 1.0)


def _k1t_first_body(we, w_syn, w_sem, deg_t3, y_txt3):
    x = we[...]
    y_txt3[0, :, :] = _dinv_txt(deg_t3[0]) * jnp.dot(
        x, w_syn[...], preferred_element_type=F32)
    y_txt3[1, :, :] = _dinv_txt(deg_t3[1]) * jnp.dot(
        x, w_sem[...], preferred_element_type=F32)


def _k1t_first(we, w_syn, w_sem, deg_t3):
    return pl.pallas_call(
        _k1t_first_body,
        grid=(NTB,),
        in_specs=[
            pl.BlockSpec((TM, 300), lambda i: (i, 0)),
            pl.BlockSpec((300, D), lambda i: (0, 0)),
            pl.BlockSpec((300, D), lambda i: (0, 0)),
            pl.BlockSpec((2, TM, 16), lambda i: (0, i, 0)),
        ],
        out_specs=pl.BlockSpec((2, TM, D), lambda i: (0, i, 0)),
        out_shape=jax.ShapeDtypeStruct((2, NT, D), F32),
    )(we, w_syn, w_sem, deg_t3)


def _k1t_mid_body(z_syn, y_syn_prev, ctx_in, b_syn_prev, w_syn, w_sem,
                  deg_t3, y_txt3):
    dv_syn = _dinv_txt(deg_t3[0])
    x_syn = dv_syn * (z_syn[0] + z_syn[1] + y_syn_prev[0]) + b_syn_prev[...]
    y_txt3[0, :, :] = dv_syn * jnp.dot(x_syn, w_syn[...],
                                       preferred_element_type=F32)
    y_txt3[1, :, :] = _dinv_txt(deg_t3[1]) * jnp.dot(
        ctx_in[...], w_sem[...], preferred_element_type=F32)


def _k1t_mid(z_txt3, y_txt3_prev, ctx_in, b_syn_prev, w_syn, w_sem, deg_t3):
    return pl.pallas_call(
        _k1t_mid_body,
        grid=(NTB,),
        in_specs=[
            pl.BlockSpec((2, TM, D), lambda i: (0, i, 0)),
            pl.BlockSpec((1, TM, D), lambda i: (0, i, 0)),
            pl.BlockSpec((TM, D), lambda i: (i, 0)),
            pl.BlockSpec((1, D), lambda i: (0, 0)),
            pl.BlockSpec((D, D), lambda i: (0, 0)),
            pl.BlockSpec((D, D), lambda i: (0, 0)),
            pl.BlockSpec((2, TM, 16), lambda i: (0, i, 0)),
        ],
        out_specs=pl.BlockSpec((2, TM, D), lambda i: (0, i, 0)),
        out_shape=jax.ShapeDtypeStruct((2, NT, D), F32),
    )(z_txt3, y_txt3_prev, ctx_in, b_syn_prev, w_syn, w_sem, deg_t3)


def _k1o_body(obj_in, w_obj, deg_o, y_obj):
    dv_obj = lax.rsqrt(deg_o[0:NO, 0:1] + deg_o[NO:2 * NO, 0:1] + 1.0)
    y_obj[...] = dv_obj * jnp.dot(obj_in[...], w_obj[...],
                                  preferred_element_type=F32)


def _k1o(obj_in, w_obj, deg_o):
    return pl.pallas_call(
        _k1o_body,
        out_shape=jax.ShapeDtypeStruct((NO, D), F32),
    )(obj_in, w_obj, deg_o)


def _k3_body(z_ctx, y_ctx, deg_c, b_sem, z_obj, y_obj, deg_o, b_obj, wk, wq,
             ctx_out, obj_out, obj_gcn_s, kk_s, acc_s, m_s, s_s):
    i = pl.program_id(0)

    @pl.when(i == 0)
    def _init():
        dv_o = lax.rsqrt(deg_o[0:NO, 0:1] + deg_o[NO:2 * NO, 0:1] + 1.0)
        og = dv_o * (z_obj[0:NO, :] + z_obj[NO:2 * NO, :] + y_obj[...]) \
            + b_obj[...]
        obj_gcn_s[...] = og
        kk_s[...] = jnp.dot(og, wk[...], preferred_element_type=F32) \
            * (1.0 / jnp.sqrt(jnp.float32(D)))
        m_s[...] = jnp.full((NO, D), -1e30, F32)
        s_s[...] = jnp.zeros((NO, D), F32)
        acc_s[...] = jnp.zeros((NO, D), F32)

    dv_c = _dinv_txt(deg_c[0])
    ctx_gcn = dv_c * (z_ctx[0] + z_ctx[1] + y_ctx[0]) + b_sem[...]
    qq = jnp.dot(ctx_gcn, wq[...], preferred_element_type=F32)
    logits = lax.dot_general(kk_s[...], qq, (((1,), (1,)), ((), ())),
                             preferred_element_type=F32)
    m_run = m_s[...]
    m_step = jnp.max(logits, axis=1, keepdims=True)
    m_new = jnp.maximum(m_run, jnp.broadcast_to(m_step, (NO, D)))
    e1 = jnp.exp(logits - m_new[:, 0:1])
    big_m = jnp.max(m_new)
    v = jnp.exp(m_new[:, 0:1] - big_m)
    g = e1 * v
    s_c = jnp.sum(g, axis=0, keepdims=True)
    ctx_out[...] = lax.dot_general(g / s_c, obj_gcn_s[...],
                                   (((0,), (0,)), ((), ())),
                                   preferred_element_type=F32)
    alpha = jnp.exp(m_run - m_new)
    r_step = jnp.sum(e1, axis=1, keepdims=True)
    s_s[...] = s_s[...] * alpha + jnp.broadcast_to(r_step, (NO, D))
    acc_s[...] = acc_s[...] * alpha + jnp.dot(e1, ctx_gcn,
                                              preferred_element_type=F32)
    m_s[...] = m_new

    @pl.when(i == NSTEP - 1)
    def _fin():
        obj_out[...] = acc_s[...] / s_s[...]


def _k3(z_txt3, y_txt3, deg_t3, b_sem, z_obj, y_obj, deg_o, b_obj, wk, wq):
    return pl.pallas_call(
        _k3_body,
        grid=(NSTEP,),
        in_specs=[
            pl.BlockSpec((2, TQ, D), lambda i: (0, i, 0)),
            pl.BlockSpec((1, TQ, D), lambda i: (1, i, 0)),
            pl.BlockSpec((1, TQ, 16), lambda i: (1, i, 0)),
            pl.BlockSpec((1, D), lambda i: (0, 0)),
            pl.BlockSpec((2 * NO, D), lambda i: (0, 0)),
            pl.BlockSpec((NO, D), lambda i: (0, 0)),
            pl.BlockSpec((2 * NO, 16), lambda i: (0, 0)),
            pl.BlockSpec((1, D), lambda i: (0, 0)),
            pl.BlockSpec((D, D), lambda i: (0, 0)),
            pl.BlockSpec((D, D), lambda i: (0, 0)),
        ],
        out_specs=(pl.BlockSpec((TQ, D), lambda i: (i, 0)),
                   pl.BlockSpec((NO, D), lambda i: (0, 0))),
        out_shape=(jax.ShapeDtypeStruct((NT, D), F32),
                   jax.ShapeDtypeStruct((NO, D), F32)),
        scratch_shapes=[
            pltpu.VMEM((NO, D), F32),
            pltpu.VMEM((NO, D), F32),
            pltpu.VMEM((NO, D), F32),
            pltpu.VMEM((NO, D), F32),
            pltpu.VMEM((NO, D), F32),
        ],
    )(z_txt3, y_txt3, deg_t3, b_sem, z_obj, y_obj, deg_o, b_obj, wk, wq)


def _k4_body(z_syn, y_syn, deg_t, b_syn, ctx_fin, obj_fin, tb, ob,
             wf1, bf1, wf2, bf2, out):
    dv_syn = _dinv_txt(deg_t[0])
    syn_fin = dv_syn * (z_syn[0] + z_syn[1] + y_syn[0]) + b_syn[...]
    iot_t = lax.broadcasted_iota(jnp.int32, (NB, NT), 0)
    oh_t = (tb[...] == iot_t).astype(F32)
    ct = jnp.maximum(jnp.sum(oh_t, axis=1, keepdims=True), 1.0)
    iot_o = lax.broadcasted_iota(jnp.int32, (NB, NO), 0)
    oh_o = (ob[...] == iot_o).astype(F32)
    co = jnp.maximum(jnp.sum(oh_o, axis=1, keepdims=True), 1.0)
    syn_p = jnp.dot(oh_t, syn_fin, preferred_element_type=F32) / ct
    ctx_p = jnp.dot(oh_t, ctx_fin[...], preferred_element_type=F32) / ct
    obj_p = jnp.dot(oh_o, obj_fin[...], preferred_element_type=F32) / co
    fused = jnp.concatenate([syn_p, obj_p, ctx_p], axis=1)
    h = jnp.dot(fused, wf1[...], preferred_element_type=F32) + bf1[...]
    lg = jnp.dot(h, wf2[...], preferred_element_type=F32) + bf2[...]
    mm = jnp.max(lg, axis=1, keepdims=True)
    out[...] = (lg - mm) - jnp.log(
        jnp.sum(jnp.exp(lg - mm), axis=1, keepdims=True))


def _k4(z_txt, y_txt, deg_t, b_syn, ctx_fin, obj_fin, tb, ob,
        wf1, bf1, wf2, bf2):
    hid = wf1.shape[1]
    return pl.pallas_call(
        _k4_body,
        grid=(1,),
        in_specs=[
            pl.BlockSpec((2, NT, D), lambda i: (0, 0, 0)),
            pl.BlockSpec((1, NT, D), lambda i: (0, 0, 0)),
            pl.BlockSpec((1, NT, 16), lambda i: (0, 0, 0)),
            pl.BlockSpec((1, D), lambda i: (0, 0)),
            pl.BlockSpec((NT, D), lambda i: (0, 0)),
            pl.BlockSpec((NO, D), lambda i: (0, 0)),
            pl.BlockSpec((1, NT), lambda i: (0, 0)),
            pl.BlockSpec((1, NO), lambda i: (0, 0)),
            pl.BlockSpec((3 * D, hid), lambda i: (0, 0)),
            pl.BlockSpec((1, hid), lambda i: (0, 0)),
            pl.BlockSpec((hid, NA), lambda i: (0, 0)),
            pl.BlockSpec((1, NA), lambda i: (0, 0)),
        ],
        out_specs=pl.BlockSpec((NB, NA), lambda i: (0, 0)),
        out_shape=jax.ShapeDtypeStruct((NB, NA), F32),
    )(z_txt, y_txt, deg_t, b_syn, ctx_fin, obj_fin, tb, ob,
      wf1, bf1, wf2, bf2)


def kernel(word_embeddings, objects, syn_edge_index, ctx_edge_index,
           obj_edge_index, txt_batch, obj_batch, params):
    p = params
    syn_src = syn_edge_index[0].astype(jnp.int32)
    syn_dst = syn_edge_index[1].astype(jnp.int32)
    ctx_src = ctx_edge_index[0].astype(jnp.int32)
    ctx_dst = ctx_edge_index[1].astype(jnp.int32)
    tdst = jnp.pad(jnp.stack([syn_dst, ctx_dst]).reshape(2, CT, 128),
                   ((0, 0), (0, CTP - CT), (0, 0)))
    npad = CTG64 - CT64
    trash = NT + (jnp.arange(npad * 64, dtype=jnp.int32)
                  % TRASH).reshape(npad, 64)
    ssrc64 = jnp.pad(syn_src.reshape(CT64, 64), ((0, npad), (0, 0)))
    sdst64 = jnp.concatenate([syn_dst.reshape(CT64, 64), trash])
    csrc64 = jnp.pad(ctx_src.reshape(CT64, 64), ((0, npad), (0, 0))) + NT
    cdst64 = jnp.concatenate([ctx_dst.reshape(CT64, 64), trash])
    osrc = obj_edge_index[0].astype(jnp.int32).reshape(CO, 128)
    odst = obj_edge_index[1].astype(jnp.int32).reshape(CO, 128)
    ones16 = jnp.ones((128, 16), F32)
    zz16 = jnp.zeros((624, 16), F32)
    zz = jnp.zeros((624, D), F32)
    tb = txt_batch.astype(jnp.int32).reshape(1, NT)
    ob = obj_batch.astype(jnp.int32).reshape(1, NO)

    sc_deg, sc_scat_t, sc_scat_o = _sc_kernels()
    deg_t, deg_o = sc_deg(tdst, odst, ones16, zz16)
    deg_t3 = deg_t.reshape(2, NT, 16)

    ctx_in = None
    obj_in = None
    y_txt3 = y_obj = z_syn2 = z_ctx2 = z_obj = None
    for l in range(3):
        if l == 0:
            y_txt3 = _k1t_first(word_embeddings, p['W_syn_0'], p['W_sem_0'],
                                deg_t3)
            y_obj = _k1o(objects, p['W_obj_0'], deg_o)
        else:
            y_txt3 = _k1t_mid(z_syn2, y_txt3, ctx_in,
                              p['b_syn_%d' % (l - 1)].reshape(1, D),
                              p['W_syn_%d' % l], p['W_sem_%d' % l], deg_t3)
            y_obj = _k1o(obj_in, p['W_obj_%d' % l], deg_o)
        y_flat = y_txt3.reshape(2 * NT, D)
        z_ctx2 = sc_scat_t(y_flat, csrc64, cdst64, zz).reshape(2, NT, D)
        z_obj = sc_scat_o(y_obj, osrc, odst, zz)
        z_syn2 = sc_scat_t(y_flat, ssrc64, sdst64, zz).reshape(2, NT, D)
        ctx_in, obj_in = _k3(
            z_ctx2, y_txt3, deg_t3, p['b_sem_%d' % l].reshape(1, D),
            z_obj, y_obj, deg_o, p['b_obj_%d' % l].reshape(1, D),
            p['Wk_%d' % l], p['Wq_%d' % l])

    hid = p['Wf1'].shape[1]
    return _k4(z_syn2, y_txt3, deg_t3, p['b_syn_2'].reshape(1, D),
               ctx_in, obj_in, tb, ob,
               p['Wf1'], p['bf1'].reshape(1, hid),
               p['Wf2'], p['bf2'].reshape(1, NA))


# speedup vs baseline: 1.7165x; 1.2414x over previous
import functools
import jax
import jax.numpy as jnp
from jax import lax
from jax.experimental import pallas as pl
from jax.experimental.pallas import tpu as pltpu
from jax.experimental.pallas import tpu_sc as plsc

NT = 10000
NO = 2048
ET = 160000
EO = 65536
NB = 64
NA = 3129
D = 128
CT = ET // 128
CTP = 1256
CT64 = ET // 64
CTG64 = 2512
NGG = 157
TRASH = 64
CO = EO // 128
TPT = 88
TQ = 1000
NSTEP = NT // TQ

F32 = jnp.float32


def _deg_body(tdst, odst, ones16, zz16, deg_t, deg_o, tidx, oidx, ones_v,
              acc_t, acc_o):
    c = lax.axis_index("c")
    s = lax.axis_index("s")
    pltpu.sync_copy(zz16, acc_t.at[pl.ds(s * 624, 624)])
    pltpu.sync_copy(zz16.at[pl.ds(0, 128)], acc_o.at[pl.ds(s * 128, 128)])

    @pl.when(s == 15)
    def _ztail():
        pltpu.sync_copy(zz16.at[pl.ds(0, 16)], acc_t.at[pl.ds(9984, 16)])

    pltpu.sync_copy(ones16, ones_v)
    plsc.subcore_barrier()
    lo = s * CT // 16
    hi = (s + 1) * CT // 16
    lo8 = (lo // 8) * 8
    pltpu.sync_copy(tdst.at[c, pl.ds(lo8, TPT)], tidx)
    pltpu.sync_copy(odst.at[pl.ds(c * (CO // 2) + s * 16, 16)], oidx)

    def tb(j, carry):
        pltpu.sync_copy(ones_v, acc_t.at[tidx.at[j]], add=True)
        return carry

    lax.fori_loop(lo - lo8, hi - lo8, tb, 0)

    def ob(j, carry):
        pltpu.sync_copy(ones_v, acc_o.at[oidx.at[j]], add=True)
        return carry

    lax.fori_loop(0, 16, ob, 0)
    plsc.subcore_barrier()
    pltpu.sync_copy(acc_t.at[pl.ds(s * 624, 624)],
                    deg_t.at[pl.ds(c * NT + s * 624, 624)])
    pltpu.sync_copy(acc_o.at[pl.ds(s * 128, 128)],
                    deg_o.at[pl.ds(c * NO + s * 128, 128)])

    @pl.when(s == 15)
    def _otail():
        pltpu.sync_copy(acc_t.at[pl.ds(9984, 16)],
                        deg_t.at[pl.ds(c * NT + 9984, 16)])


def _scatter_txt_body(y_txt, tsrc, tdst, zz, z2,
                      tsrc_v, tdst_v, r0, r1, r2, r3, acc_t,
                      sg0, sg1, sg2, sg3, ss0, ss1, ss2, ss3):
    c = lax.axis_index("c")
    s = lax.axis_index("s")
    pltpu.sync_copy(zz, acc_t.at[pl.ds(s * 624, 624)])

    @pl.when(s == 15)
    def _ztail():
        pltpu.sync_copy(zz.at[pl.ds(0, 16)], acc_t.at[pl.ds(9984, 16)])

    plsc.subcore_barrier()
    g_lo = s * NGG // 16
    g_hi = (s + 1) * NGG // 16
    bufs = [(r0, sg0, ss0), (r1, sg1, ss1), (r2, sg2, ss2), (r3, sg3, ss3)]

    def grp(g, carry):
        base = (c * NGG + g) * 8
        pltpu.sync_copy(tsrc.at[pl.ds(base, 8)], tsrc_v)
        pltpu.sync_copy(tdst.at[pl.ds(base, 8)], tdst_v)
        for t in range(10):
            if t < 8:
                buf, sg, ss = bufs[t % 4]
                if t >= 4:
                    pltpu.make_async_copy(y_txt.at[pl.ds(0, 64)], buf,
                                          ss).wait()
                pltpu.async_copy(y_txt.at[tsrc_v.at[t]], buf, sg)
            if 2 <= t:
                j = t - 2
                buf, sg, ss = bufs[j % 4]
                pltpu.make_async_copy(y_txt.at[pl.ds(0, 64)], buf, sg).wait()
                pltpu.async_copy(buf, acc_t.at[tdst_v.at[j]], ss, add=True)
        for b in range(4):
            pltpu.make_async_copy(y_txt.at[pl.ds(0, 64)], bufs[b][0],
                                  bufs[b][2]).wait()
        return carry

    lax.fori_loop(g_lo, g_hi, grp, 0)
    plsc.subcore_barrier()
    pltpu.sync_copy(acc_t.at[pl.ds(s * 624, 624)],
                    z2.at[pl.ds(c * NT + s * 624, 624)])

    @pl.when(s == 15)
    def _otail():
        pltpu.sync_copy(acc_t.at[pl.ds(9984, 16)],
                        z2.at[pl.ds(c * NT + 9984, 16)])


def _scatter_obj_body(y_obj, osrc, odst, zz, z_obj,
                      osrc_v, odst_v, rows_a, rows_b, acc_o,
                      sga, sgb, ssa, ssb):
    c = lax.axis_index("c")
    s = lax.axis_index("s")
    pltpu.sync_copy(zz.at[pl.ds(0, 128)], acc_o.at[pl.ds(s * 128, 128)])
    plsc.subcore_barrier()
    pltpu.sync_copy(osrc.at[pl.ds(c * (CO // 2) + s * 16, 16)], osrc_v)
    pltpu.sync_copy(odst.at[pl.ds(c * (CO // 2) + s * 16, 16)], odst_v)

    def fire(j, buf, sm):
        pltpu.async_copy(y_obj.at[osrc_v.at[j]], buf, sm)

    def wait(buf, sm):
        pltpu.make_async_copy(y_obj.at[pl.ds(0, 128)], buf, sm).wait()

    fire(0, rows_a, sga)
    fire(1, rows_b, sgb)

    def ob(k, carry):
        j0 = 2 * k
        wait(rows_a, sga)
        pltpu.async_copy(rows_a, acc_o.at[odst_v.at[j0]], ssa, add=True)
        wait(rows_b, sgb)
        pltpu.async_copy(rows_b, acc_o.at[odst_v.at[j0 + 1]], ssb, add=True)
        wait(rows_a, ssa)

        @pl.when(2 * k + 2 < 16)
        def _ra():
            fire(j0 + 2, rows_a, sga)

        wait(rows_b, ssb)

        @pl.when(2 * k + 3 < 16)
        def _rb():
            fire(j0 + 3, rows_b, sgb)

        return carry

    lax.fori_loop(0, 8, ob, 0)
    plsc.subcore_barrier()
    pltpu.sync_copy(acc_o.at[pl.ds(s * 128, 128)],
                    z_obj.at[pl.ds(c * NO + s * 128, 128)])


@functools.cache
def _sc_kernels():
    mesh = plsc.VectorSubcoreMesh(core_axis_name="c", subcore_axis_name="s",
                                  num_cores=2, num_subcores=16)
    deg = pl.kernel(
        _deg_body,
        out_type=(jax.ShapeDtypeStruct((2 * NT, 16), F32),
                  jax.ShapeDtypeStruct((2 * NO, 16), F32)),
        mesh=mesh,
        scratch_types=[
            pltpu.VMEM((TPT, 128), jnp.int32),
            pltpu.VMEM((16, 128), jnp.int32),
            pltpu.VMEM((128, 16), F32),
            pltpu.VMEM_SHARED((NT, 16), F32),
            pltpu.VMEM_SHARED((NO, 16), F32),
        ],
    )
    scat_t = pl.kernel(
        _scatter_txt_body,
        out_type=jax.ShapeDtypeStruct((2 * NT, D), F32),
        mesh=mesh,
        scratch_types=[
            pltpu.VMEM((8, 64), jnp.int32),
            pltpu.VMEM((8, 64), jnp.int32),
            pltpu.VMEM((64, D), F32),
            pltpu.VMEM((64, D), F32),
            pltpu.VMEM((64, D), F32),
            pltpu.VMEM((64, D), F32),
            pltpu.VMEM_SHARED((NT + TRASH, D), F32),
        ] + [pltpu.SemaphoreType.DMA] * 8,
    )
    scat_o = pl.kernel(
        _scatter_obj_body,
        out_type=jax.ShapeDtypeStruct((2 * NO, D), F32),
        mesh=mesh,
        scratch_types=[
            pltpu.VMEM((16, 128), jnp.int32),
            pltpu.VMEM((16, 128), jnp.int32),
            pltpu.VMEM((128, D), F32),
            pltpu.VMEM((128, D), F32),
            pltpu.VMEM_SHARED((NO, D), F32),
            pltpu.SemaphoreType.DMA,
            pltpu.SemaphoreType.DMA,
            pltpu.SemaphoreType.DMA,
            pltpu.SemaphoreType.DMA,
        ],
    )
    return deg, scat_t, scat_o


TM = 2000
NTB = NT // TM


def _dinv_txt(deg_blk):
    return lax.rsqrt(deg_blk[:, 0:1] + 1.0)


def _k1t_first_body(we, w_syn, w_sem, deg_t3, y_txt3):
    x = we[...]
    y_txt3[0, :, :] = _dinv_txt(deg_t3[0]) * jnp.dot(
        x, w_syn[...], preferred_element_type=F32)
    y_txt3[1, :, :] = _dinv_txt(deg_t3[1]) * jnp.dot(
        x, w_sem[...], preferred_element_type=F32)


def _k1t_first(we, w_syn, w_sem, deg_t3):
    return pl.pallas_call(
        _k1t_first_body,
        grid=(NTB,),
        in_specs=[
            pl.BlockSpec((TM, 300), lambda i: (i, 0)),
            pl.BlockSpec((300, D), lambda i: (0, 0)),
            pl.BlockSpec((300, D), lambda i: (0, 0)),
            pl.BlockSpec((2, TM, 16), lambda i: (0, i, 0)),
        ],
        out_specs=pl.BlockSpec((2, TM, D), lambda i: (0, i, 0)),
        out_shape=jax.ShapeDtypeStruct((2, NT, D), F32),
    )(we, w_syn, w_sem, deg_t3)


def _k1t_mid_body(z_syn, y_syn_prev, ctx_in, b_syn_prev, w_syn, w_sem,
                  deg_t3, y_txt3):
    dv_syn = _dinv_txt(deg_t3[0])
    x_syn = dv_syn * (z_syn[0] + z_syn[1] + y_syn_prev[0]) + b_syn_prev[...]
    y_txt3[0, :, :] = dv_syn * jnp.dot(x_syn, w_syn[...],
                                       preferred_element_type=F32)
    y_txt3[1, :, :] = _dinv_txt(deg_t3[1]) * jnp.dot(
        ctx_in[...], w_sem[...], preferred_element_type=F32)


def _k1t_mid(z_txt3, y_txt3_prev, ctx_in, b_syn_prev, w_syn, w_sem, deg_t3):
    return pl.pallas_call(
        _k1t_mid_body,
        grid=(NTB,),
        in_specs=[
            pl.BlockSpec((2, TM, D), lambda i: (0, i, 0)),
            pl.BlockSpec((1, TM, D), lambda i: (0, i, 0)),
            pl.BlockSpec((TM, D), lambda i: (i, 0)),
            pl.BlockSpec((1, D), lambda i: (0, 0)),
            pl.BlockSpec((D, D), lambda i: (0, 0)),
            pl.BlockSpec((D, D), lambda i: (0, 0)),
            pl.BlockSpec((2, TM, 16), lambda i: (0, i, 0)),
        ],
        out_specs=pl.BlockSpec((2, TM, D), lambda i: (0, i, 0)),
        out_shape=jax.ShapeDtypeStruct((2, NT, D), F32),
    )(z_txt3, y_txt3_prev, ctx_in, b_syn_prev, w_syn, w_sem, deg_t3)


def _k1o_body(obj_in, w_obj, deg_o, y_obj):
    dv_obj = lax.rsqrt(deg_o[0:NO, 0:1] + deg_o[NO:2 * NO, 0:1] + 1.0)
    y_obj[...] = dv_obj * jnp.dot(obj_in[...], w_obj[...],
                                  preferred_element_type=F32)


def _k1o(obj_in, w_obj, deg_o):
    return pl.pallas_call(
        _k1o_body,
        out_shape=jax.ShapeDtypeStruct((NO, D), F32),
    )(obj_in, w_obj, deg_o)


def _k3_body(z_ctx, y_ctx, deg_c, b_sem, z_obj, y_obj, deg_o, b_obj, wk, wq,
             ctx_out, obj_out, obj_gcn_s, kk_s, acc_s, m_s, s_s):
    i = pl.program_id(0)

    @pl.when(i == 0)
    def _init():
        dv_o = lax.rsqrt(deg_o[0:NO, 0:1] + deg_o[NO:2 * NO, 0:1] + 1.0)
        og = dv_o * (z_obj[0:NO, :] + z_obj[NO:2 * NO, :] + y_obj[...]) \
            + b_obj[...]
        obj_gcn_s[...] = og
        kk_s[...] = jnp.dot(og, wk[...], preferred_element_type=F32) \
            * (1.0 / jnp.sqrt(jnp.float32(D)))
        m_s[...] = jnp.full((NO, D), -1e30, F32)
        s_s[...] = jnp.zeros((NO, D), F32)
        acc_s[...] = jnp.zeros((NO, D), F32)

    dv_c = _dinv_txt(deg_c[0])
    ctx_gcn = dv_c * (z_ctx[0] + z_ctx[1] + y_ctx[0]) + b_sem[...]
    qq = jnp.dot(ctx_gcn, wq[...], preferred_element_type=F32)
    logits = lax.dot_general(kk_s[...], qq, (((1,), (1,)), ((), ())),
                             preferred_element_type=F32)
    m_run = m_s[...]
    m_step = jnp.max(logits, axis=1, keepdims=True)
    m_new = jnp.maximum(m_run, jnp.broadcast_to(m_step, (NO, D)))
    e1 = jnp.exp(logits - m_new[:, 0:1])
    big_m = jnp.max(m_new)
    v = jnp.exp(m_new[:, 0:1] - big_m)
    g = e1 * v
    s_c = jnp.sum(g, axis=0, keepdims=True)
    ctx_out[...] = lax.dot_general(g / s_c, obj_gcn_s[...],
                                   (((0,), (0,)), ((), ())),
                                   preferred_element_type=F32)
    alpha = jnp.exp(m_run - m_new)
    r_step = jnp.sum(e1, axis=1, keepdims=True)
    s_s[...] = s_s[...] * alpha + jnp.broadcast_to(r_step, (NO, D))
    acc_s[...] = acc_s[...] * alpha + jnp.dot(e1, ctx_gcn,
                                              preferred_element_type=F32)
    m_s[...] = m_new

    @pl.when(i == NSTEP - 1)
    def _fin():
        obj_out[...] = acc_s[...] / s_s[...]


def _k3(z_txt3, y_txt3, deg_t3, b_sem, z_obj, y_obj, deg_o, b_obj, wk, wq):
    return pl.pallas_call(
        _k3_body,
        grid=(NSTEP,),
        in_specs=[
            pl.BlockSpec((2, TQ, D), lambda i: (0, i, 0)),
            pl.BlockSpec((1, TQ, D), lambda i: (1, i, 0)),
            pl.BlockSpec((1, TQ, 16), lambda i: (1, i, 0)),
            pl.BlockSpec((1, D), lambda i: (0, 0)),
            pl.BlockSpec((2 * NO, D), lambda i: (0, 0)),
            pl.BlockSpec((NO, D), lambda i: (0, 0)),
            pl.BlockSpec((2 * NO, 16), lambda i: (0, 0)),
            pl.BlockSpec((1, D), lambda i: (0, 0)),
            pl.BlockSpec((D, D), lambda i: (0, 0)),
            pl.BlockSpec((D, D), lambda i: (0, 0)),
        ],
        out_specs=(pl.BlockSpec((TQ, D), lambda i: (i, 0)),
                   pl.BlockSpec((NO, D), lambda i: (0, 0))),
        out_shape=(jax.ShapeDtypeStruct((NT, D), F32),
                   jax.ShapeDtypeStruct((NO, D), F32)),
        scratch_shapes=[
            pltpu.VMEM((NO, D), F32),
            pltpu.VMEM((NO, D), F32),
            pltpu.VMEM((NO, D), F32),
            pltpu.VMEM((NO, D), F32),
            pltpu.VMEM((NO, D), F32),
        ],
    )(z_txt3, y_txt3, deg_t3, b_sem, z_obj, y_obj, deg_o, b_obj, wk, wq)


def _k4_body(z_syn, y_syn, deg_t, b_syn, ctx_fin, obj_fin, tb, ob,
             wf1, bf1, wf2, bf2, out):
    dv_syn = _dinv_txt(deg_t[0])
    syn_fin = dv_syn * (z_syn[0] + z_syn[1] + y_syn[0]) + b_syn[...]
    iot_t = lax.broadcasted_iota(jnp.int32, (NB, NT), 0)
    oh_t = (tb[...] == iot_t).astype(F32)
    ct = jnp.maximum(jnp.sum(oh_t, axis=1, keepdims=True), 1.0)
    iot_o = lax.broadcasted_iota(jnp.int32, (NB, NO), 0)
    oh_o = (ob[...] == iot_o).astype(F32)
    co = jnp.maximum(jnp.sum(oh_o, axis=1, keepdims=True), 1.0)
    syn_p = jnp.dot(oh_t, syn_fin, preferred_element_type=F32) / ct
    ctx_p = jnp.dot(oh_t, ctx_fin[...], preferred_element_type=F32) / ct
    obj_p = jnp.dot(oh_o, obj_fin[...], preferred_element_type=F32) / co
    fused = jnp.concatenate([syn_p, obj_p, ctx_p], axis=1)
    h = jnp.dot(fused, wf1[...], preferred_element_type=F32) + bf1[...]
    lg = jnp.dot(h, wf2[...], preferred_element_type=F32) + bf2[...]
    mm = jnp.max(lg, axis=1, keepdims=True)
    out[...] = (lg - mm) - jnp.log(
        jnp.sum(jnp.exp(lg - mm), axis=1, keepdims=True))


def _k4(z_txt, y_txt, deg_t, b_syn, ctx_fin, obj_fin, tb, ob,
        wf1, bf1, wf2, bf2):
    hid = wf1.shape[1]
    return pl.pallas_call(
        _k4_body,
        grid=(1,),
        in_specs=[
            pl.BlockSpec((2, NT, D), lambda i: (0, 0, 0)),
            pl.BlockSpec((1, NT, D), lambda i: (0, 0, 0)),
            pl.BlockSpec((1, NT, 16), lambda i: (0, 0, 0)),
            pl.BlockSpec((1, D), lambda i: (0, 0)),
            pl.BlockSpec((NT, D), lambda i: (0, 0)),
            pl.BlockSpec((NO, D), lambda i: (0, 0)),
            pl.BlockSpec((1, NT), lambda i: (0, 0)),
            pl.BlockSpec((1, NO), lambda i: (0, 0)),
            pl.BlockSpec((3 * D, hid), lambda i: (0, 0)),
            pl.BlockSpec((1, hid), lambda i: (0, 0)),
            pl.BlockSpec((hid, NA), lambda i: (0, 0)),
            pl.BlockSpec((1, NA), lambda i: (0, 0)),
        ],
        out_specs=pl.BlockSpec((NB, NA), lambda i: (0, 0)),
        out_shape=jax.ShapeDtypeStruct((NB, NA), F32),
    )(z_txt, y_txt, deg_t, b_syn, ctx_fin, obj_fin, tb, ob,
      wf1, bf1, wf2, bf2)


def kernel(word_embeddings, objects, syn_edge_index, ctx_edge_index,
           obj_edge_index, txt_batch, obj_batch, params):
    p = params
    syn_src = syn_edge_index[0].astype(jnp.int32)
    syn_dst = syn_edge_index[1].astype(jnp.int32)
    ctx_src = ctx_edge_index[0].astype(jnp.int32)
    ctx_dst = ctx_edge_index[1].astype(jnp.int32)
    tdst = jnp.pad(jnp.stack([syn_dst, ctx_dst]).reshape(2, CT, 128),
                   ((0, 0), (0, CTP - CT), (0, 0)))
    npad = CTG64 - CT64
    spread = (jnp.arange(npad * 64, dtype=jnp.int32)
              % TRASH).reshape(npad, 64)
    ssrc64 = jnp.concatenate([syn_src.reshape(CT64, 64), spread])
    sdst64 = jnp.concatenate([syn_dst.reshape(CT64, 64), NT + spread])
    csrc64 = jnp.concatenate([ctx_src.reshape(CT64, 64) + NT, NT + spread])
    cdst64 = jnp.concatenate([ctx_dst.reshape(CT64, 64), NT + spread])
    osrc = obj_edge_index[0].astype(jnp.int32).reshape(CO, 128)
    odst = obj_edge_index[1].astype(jnp.int32).reshape(CO, 128)
    ones16 = jnp.ones((128, 16), F32)
    zz16 = jnp.zeros((624, 16), F32)
    zz = jnp.zeros((624, D), F32)
    tb = txt_batch.astype(jnp.int32).reshape(1, NT)
    ob = obj_batch.astype(jnp.int32).reshape(1, NO)

    sc_deg, sc_scat_t, sc_scat_o = _sc_kernels()
    deg_t, deg_o = sc_deg(tdst, odst, ones16, zz16)
    deg_t3 = deg_t.reshape(2, NT, 16)

    ctx_in = None
    obj_in = None
    y_txt3 = y_obj = z_syn2 = z_ctx2 = z_obj = None
    for l in range(3):
        if l == 0:
            y_txt3 = _k1t_first(word_embeddings, p['W_syn_0'], p['W_sem_0'],
                                deg_t3)
            y_obj = _k1o(objects, p['W_obj_0'], deg_o)
        else:
            y_txt3 = _k1t_mid(z_syn2, y_txt3, ctx_in,
                              p['b_syn_%d' % (l - 1)].reshape(1, D),
                              p['W_syn_%d' % l], p['W_sem_%d' % l], deg_t3)
            y_obj = _k1o(obj_in, p['W_obj_%d' % l], deg_o)
        y_flat = y_txt3.reshape(2 * NT, D)
        z_ctx2 = sc_scat_t(y_flat, csrc64, cdst64, zz).reshape(2, NT, D)
        z_obj = sc_scat_o(y_obj, osrc, odst, zz)
        z_syn2 = sc_scat_t(y_flat, ssrc64, sdst64, zz).reshape(2, NT, D)
        ctx_in, obj_in = _k3(
            z_ctx2, y_txt3, deg_t3, p['b_sem_%d' % l].reshape(1, D),
            z_obj, y_obj, deg_o, p['b_obj_%d' % l].reshape(1, D),
            p['Wk_%d' % l], p['Wq_%d' % l])

    hid = p['Wf1'].shape[1]
    return _k4(z_syn2, y_txt3, deg_t3, p['b_syn_2'].reshape(1, D),
               ctx_in, obj_in, tb, ob,
               p['Wf1'], p['bf1'].reshape(1, hid),
               p['Wf2'], p['bf2'].reshape(1, NA))
